# baseline (speedup 1.0000x reference)
import functools

import jax
import jax.numpy as jnp
from jax import lax
from jax.experimental import pallas as pl
from jax.experimental.pallas import tpu as pltpu
from jax.experimental.pallas import tpu_sc as plsc

NN = 10000
EE = 320000
FIN = 128
HID = 256
CC = 64

P = 10240
ROWS = P // 16
K = 128
NB = 160
EP = 16 * NB * K
NBD = 80
NBUF = 5
BLK = 256


@functools.cache
def _mesh():
    return plsc.VectorSubcoreMesh(core_axis_name="c", subcore_axis_name="s",
                                  num_cores=2, num_subcores=16)


@functools.cache
def _make_deg():
    @functools.partial(
        pl.kernel,
        out_type=jax.ShapeDtypeStruct((2, P, 16), jnp.float32),
        mesh=_mesh(),
        compiler_params=pltpu.CompilerParams(use_tc_tiling_on_sc=False),
        scratch_types=[
            pltpu.VMEM((NBD, K), jnp.int32),
            pltpu.VMEM((K, 16), jnp.float32),
            pltpu.VMEM((ROWS, 16), jnp.float32),
            pltpu.VMEM_SHARED((P, 16), jnp.float32),
        ],
    )
    def deg_kernel(dst_hbm, out_hbm, didx, ones_v, zbuf, accum):
        c = lax.axis_index("c")
        s = lax.axis_index("s")
        t = c * 16 + s
        pltpu.sync_copy(dst_hbm.at[t], didx)
        one16 = jnp.ones((16,), jnp.float32)
        zero16 = jnp.zeros((16,), jnp.float32)

        def initbufs(i, carry):
            @pl.when(i < K)
            def _():
                ones_v[i, :] = one16
            zbuf[i, :] = zero16
            return carry

        lax.fori_loop(0, ROWS, initbufs, 0)
        pltpu.sync_copy(zbuf, accum.at[pl.ds(s * ROWS, ROWS)])
        plsc.subcore_barrier()

        def body(j, carry):
            pltpu.sync_copy(ones_v, accum.at[didx.at[j]], add=True)
            return carry

        lax.fori_loop(0, NBD, body, 0)
        plsc.subcore_barrier()
        pltpu.sync_copy(accum.at[pl.ds(s * ROWS, ROWS)],
                        out_hbm.at[c, pl.ds(s * ROWS, ROWS)])

    return deg_kernel


@functools.cache
def _make_prop(F, NCH):
    per_core = NCH // 2

    @functools.partial(
        pl.kernel,
        out_type=jax.ShapeDtypeStruct((NCH * P, F), jnp.float32),
        mesh=_mesh(),
        compiler_params=pltpu.CompilerParams(use_tc_tiling_on_sc=False),
        scratch_types=[
            pltpu.VMEM((NB, K), jnp.int32),
            pltpu.VMEM((NB, K), jnp.int32),
            [pltpu.VMEM((K, F), jnp.float32)] * NBUF,
            [pltpu.SemaphoreType.DMA] * NBUF,
            pltpu.VMEM_SHARED((P, F), jnp.float32),
        ],
    )
    def prop(t_hbm, src_hbm, dst_hbm, out_hbm, sidx, didx, rows, gsems,
             accum):
        c = lax.axis_index("c")
        s = lax.axis_index("s")
        ngrp = NB // NBUF
        pltpu.sync_copy(dst_hbm.at[s], didx)
        for cc in range(per_core):
            ch = c * per_core + cc
            pltpu.sync_copy(src_hbm.at[ch, s], sidx)
            base = ch * P + s * ROWS
            pltpu.sync_copy(t_hbm.at[pl.ds(base, ROWS)],
                            accum.at[pl.ds(s * ROWS, ROWS)])
            plsc.subcore_barrier()

            for b in range(NBUF):
                pltpu.async_copy(t_hbm.at[sidx.at[b]], rows[b], gsems[b])

            def group(g, carry):
                for b in range(NBUF):
                    j = g * NBUF + b
                    pltpu.make_async_copy(t_hbm.at[sidx.at[j]],
                                          rows[b], gsems[b]).wait()
                    pltpu.sync_copy(rows[b], accum.at[didx.at[j]], add=True)

                    @pl.when(g + 1 < ngrp)
                    def _():
                        pltpu.async_copy(t_hbm.at[sidx.at[j + NBUF]],
                                         rows[b], gsems[b])
                return carry

            lax.fori_loop(0, ngrp, group, 0)
            plsc.subcore_barrier()
            pltpu.sync_copy(accum.at[pl.ds(s * ROWS, ROWS)],
                            out_hbm.at[pl.ds(base, ROWS)])

    return prop


def _mm1_body(x_ref, w_ref, degp_ref, o_ref):
    deg = degp_ref[0, :, 0:1] + degp_ref[1, :, 0:1] + 1.0
    dinv = lax.rsqrt(deg)
    h = jnp.dot(x_ref[...], w_ref[...], preferred_element_type=jnp.float32)
    hs = h * dinv
    for i in range(4):
        o_ref[i] = hs[:, 64 * i:64 * (i + 1)]


def _mm2_body(s1_ref, degp_ref, b1_ref, w2_ref, o_ref):
    deg = degp_ref[0, :, 0:1] + degp_ref[1, :, 0:1] + 1.0
    dinv = lax.rsqrt(deg)
    acc = None
    for i in range(4):
        z = jnp.maximum(s1_ref[i] * dinv + b1_ref[i], 0.0)
        zi = jnp.dot(z, w2_ref[i], preferred_element_type=jnp.float32)
        acc = zi if acc is None else acc + zi
    hs2 = acc * dinv
    o_ref[0] = hs2[:, :32]
    o_ref[1] = hs2[:, 32:]


def _lsm_body(s2_ref, degp_ref, b2_ref, o_ref):
    deg = degp_ref[0, :, 0:1] + degp_ref[1, :, 0:1] + 1.0
    dinv = lax.rsqrt(deg)
    z0 = s2_ref[0] * dinv + b2_ref[0]
    z1 = s2_ref[1] * dinv + b2_ref[1]
    m = jnp.maximum(jnp.max(z0, axis=1, keepdims=True),
                    jnp.max(z1, axis=1, keepdims=True))
    se = (jnp.sum(jnp.exp(z0 - m), axis=1, keepdims=True)
          + jnp.sum(jnp.exp(z1 - m), axis=1, keepdims=True))
    lse = m + jnp.log(se)
    o_ref[:, :32] = z0 - lse
    o_ref[:, 32:] = z1 - lse


def kernel(x, edge_index, W1, b1, W2, b2):
    src = edge_index[0].astype(jnp.int32)
    dst = edge_index[1].astype(jnp.int32)
    pad = EP - EE
    pad_idx = NN + (jnp.arange(pad, dtype=jnp.int32) % (P - NN))
    src_p = jnp.concatenate([src, pad_idx])
    dst_p = jnp.concatenate([dst, pad_idx])
    src_l1 = jnp.stack([src_p + ch * P for ch in range(4)]).reshape(4, 16, NB, K)
    src_l2 = jnp.stack([src_p + ch * P for ch in range(2)]).reshape(2, 16, NB, K)
    dst_prop = dst_p.reshape(16, NB, K)
    dst_deg = dst_p.reshape(32, NBD, K)
    x_p = jnp.pad(x, ((0, P - NN), (0, 0)))
    b1r = b1.reshape(4, 1, 64)
    b2r = b2.reshape(2, 1, 32)
    W2r = W2.reshape(4, 64, CC)

    degp = _make_deg()(dst_deg)

    t1 = pl.pallas_call(
        _mm1_body,
        grid=(P // BLK,),
        in_specs=[
            pl.BlockSpec((BLK, FIN), lambda i: (i, 0)),
            pl.BlockSpec((FIN, HID), lambda i: (0, 0)),
            pl.BlockSpec((2, BLK, 16), lambda i: (0, i, 0)),
        ],
        out_specs=pl.BlockSpec((4, BLK, 64), lambda i: (0, i, 0)),
        out_shape=jax.ShapeDtypeStruct((4, P, 64), jnp.float32),
    )(x_p, W1, degp)

    s1 = _make_prop(64, 4)(t1.reshape(4 * P, 64), src_l1, dst_prop)

    t2 = pl.pallas_call(
        _mm2_body,
        grid=(P // BLK,),
        in_specs=[
            pl.BlockSpec((4, BLK, 64), lambda i: (0, i, 0)),
            pl.BlockSpec((2, BLK, 16), lambda i: (0, i, 0)),
            pl.BlockSpec((4, 1, 64), lambda i: (0, 0, 0)),
            pl.BlockSpec((4, 64, CC), lambda i: (0, 0, 0)),
        ],
        out_specs=pl.BlockSpec((2, BLK, 32), lambda i: (0, i, 0)),
        out_shape=jax.ShapeDtypeStruct((2, P, 32), jnp.float32),
    )(s1.reshape(4, P, 64), degp, b1r, W2r)

    s2 = _make_prop(32, 2)(t2.reshape(2 * P, 32), src_l2, dst_prop)

    out = pl.pallas_call(
        _lsm_body,
        grid=(P // BLK,),
        in_specs=[
            pl.BlockSpec((2, BLK, 32), lambda i: (0, i, 0)),
            pl.BlockSpec((2, BLK, 16), lambda i: (0, i, 0)),
            pl.BlockSpec((2, 1, 32), lambda i: (0, 0, 0)),
        ],
        out_specs=pl.BlockSpec((BLK, CC), lambda i: (i, 0)),
        out_shape=jax.ShapeDtypeStruct((P, CC), jnp.float32),
    )(s2.reshape(2, P, 32), degp, b2r)

    return out[:NN]


# speedup vs baseline: 1.0304x; 1.0304x over previous
import functools

import jax
import jax.numpy as jnp
from jax import lax
from jax.experimental import pallas as pl
from jax.experimental.pallas import tpu as pltpu
from jax.experimental.pallas import tpu_sc as plsc

NN = 10000
EE = 320000
FIN = 128
HID = 256
CC = 64

P = 10240
ROWS = P // 16
K = 128
NB = 160
EP = 16 * NB * K
NBD = 80
NBUF = 5
BLK = 256


@functools.cache
def _mesh():
    return plsc.VectorSubcoreMesh(core_axis_name="c", subcore_axis_name="s",
                                  num_cores=2, num_subcores=16)


@functools.cache
def _make_deg():
    @functools.partial(
        pl.kernel,
        out_type=jax.ShapeDtypeStruct((2, P, 16), jnp.float32),
        mesh=_mesh(),
        compiler_params=pltpu.CompilerParams(use_tc_tiling_on_sc=False),
        scratch_types=[
            pltpu.VMEM((NBD, K), jnp.int32),
            pltpu.VMEM((K, 16), jnp.float32),
            pltpu.VMEM((ROWS, 16), jnp.float32),
            pltpu.VMEM_SHARED((P, 16), jnp.float32),
        ],
    )
    def deg_kernel(dst_hbm, out_hbm, didx, ones_v, zbuf, accum):
        c = lax.axis_index("c")
        s = lax.axis_index("s")
        pltpu.sync_copy(dst_hbm.at[s, pl.ds(c * NBD, NBD)], didx)
        one16 = jnp.ones((16,), jnp.float32)
        zero16 = jnp.zeros((16,), jnp.float32)

        def initbufs(i, carry):
            @pl.when(i < K)
            def _():
                ones_v[i, :] = one16
            zbuf[i, :] = zero16
            return carry

        lax.fori_loop(0, ROWS, initbufs, 0)
        pltpu.sync_copy(zbuf, accum.at[pl.ds(s * ROWS, ROWS)])
        plsc.subcore_barrier()

        def body(j, carry):
            pltpu.sync_copy(ones_v, accum.at[didx.at[j]], add=True)
            return carry

        lax.fori_loop(0, NBD, body, 0)
        plsc.subcore_barrier()
        pltpu.sync_copy(accum.at[pl.ds(s * ROWS, ROWS)],
                        out_hbm.at[c, pl.ds(s * ROWS, ROWS)])

    return deg_kernel


@functools.cache
def _make_prop(F, NCH):
    per_core = NCH // 2

    @functools.partial(
        pl.kernel,
        out_type=jax.ShapeDtypeStruct((NCH, P, F), jnp.float32),
        mesh=_mesh(),
        compiler_params=pltpu.CompilerParams(use_tc_tiling_on_sc=False),
        scratch_types=[
            pltpu.VMEM((NB, K), jnp.int32),
            pltpu.VMEM((NB, K), jnp.int32),
            [pltpu.VMEM((K, F), jnp.float32)] * NBUF,
            [pltpu.SemaphoreType.DMA] * NBUF,
            pltpu.VMEM_SHARED((P, F), jnp.float32),
        ],
    )
    def prop(t_hbm, src_hbm, dst_hbm, out_hbm, sidx, didx, rows, gsems,
             accum):
        c = lax.axis_index("c")
        s = lax.axis_index("s")
        ngrp = NB // NBUF
        pltpu.sync_copy(dst_hbm.at[s], didx)
        pltpu.sync_copy(src_hbm.at[s], sidx)
        for cc in range(per_core):
            ch = c * per_core + cc
            tch = t_hbm.at[ch]
            pltpu.sync_copy(tch.at[pl.ds(s * ROWS, ROWS)],
                            accum.at[pl.ds(s * ROWS, ROWS)])
            plsc.subcore_barrier()

            for b in range(NBUF):
                pltpu.async_copy(tch.at[sidx.at[b]], rows[b], gsems[b])

            def group(g, carry):
                for b in range(NBUF):
                    j = g * NBUF + b
                    pltpu.make_async_copy(tch.at[sidx.at[j]],
                                          rows[b], gsems[b]).wait()
                    pltpu.sync_copy(rows[b], accum.at[didx.at[j]], add=True)

                    @pl.when(g + 1 < ngrp)
                    def _():
                        pltpu.async_copy(tch.at[sidx.at[j + NBUF]],
                                         rows[b], gsems[b])
                return carry

            lax.fori_loop(0, ngrp, group, 0)
            plsc.subcore_barrier()
            pltpu.sync_copy(accum.at[pl.ds(s * ROWS, ROWS)],
                            out_hbm.at[ch, pl.ds(s * ROWS, ROWS)])

    return prop


def _mm1_body(x_ref, w_ref, degp_ref, o_ref):
    deg = degp_ref[0, :, 0:1] + degp_ref[1, :, 0:1] + 1.0
    dinv = lax.rsqrt(deg)
    h = jnp.dot(x_ref[...], w_ref[...], preferred_element_type=jnp.float32)
    hs = h * dinv
    for i in range(4):
        o_ref[i] = hs[:, 64 * i:64 * (i + 1)]


def _mm2_body(s1_ref, degp_ref, b1_ref, w2_ref, o_ref):
    deg = degp_ref[0, :, 0:1] + degp_ref[1, :, 0:1] + 1.0
    dinv = lax.rsqrt(deg)
    acc = None
    for i in range(4):
        z = jnp.maximum(s1_ref[i] * dinv + b1_ref[i], 0.0)
        zi = jnp.dot(z, w2_ref[i], preferred_element_type=jnp.float32)
        acc = zi if acc is None else acc + zi
    hs2 = acc * dinv
    o_ref[0] = hs2[:, :32]
    o_ref[1] = hs2[:, 32:]


def _lsm_body(s2_ref, degp_ref, b2_ref, o_ref):
    deg = degp_ref[0, :, 0:1] + degp_ref[1, :, 0:1] + 1.0
    dinv = lax.rsqrt(deg)
    z0 = s2_ref[0] * dinv + b2_ref[0]
    z1 = s2_ref[1] * dinv + b2_ref[1]
    m = jnp.maximum(jnp.max(z0, axis=1, keepdims=True),
                    jnp.max(z1, axis=1, keepdims=True))
    se = (jnp.sum(jnp.exp(z0 - m), axis=1, keepdims=True)
          + jnp.sum(jnp.exp(z1 - m), axis=1, keepdims=True))
    lse = m + jnp.log(se)
    o_ref[:, :32] = z0 - lse
    o_ref[:, 32:] = z1 - lse


def kernel(x, edge_index, W1, b1, W2, b2):
    src = edge_index[0].astype(jnp.int32)
    dst = edge_index[1].astype(jnp.int32)
    pad = EP - EE
    pad_idx = NN + (jnp.arange(pad, dtype=jnp.int32) % (P - NN))
    src_pr = jnp.concatenate([src, pad_idx]).reshape(16, NB, K)
    dst_pr = jnp.concatenate([dst, pad_idx]).reshape(16, NB, K)
    b1r = b1.reshape(4, 1, 64)
    b2r = b2.reshape(2, 1, 32)
    W2r = W2.reshape(4, 64, CC)

    degp = _make_deg()(dst_pr)

    t1 = pl.pallas_call(
        _mm1_body,
        grid=(NN // 400,),
        in_specs=[
            pl.BlockSpec((400, FIN), lambda i: (i, 0)),
            pl.BlockSpec((FIN, HID), lambda i: (0, 0)),
            pl.BlockSpec((2, 400, 16), lambda i: (0, i, 0)),
        ],
        out_specs=pl.BlockSpec((4, 400, 64), lambda i: (0, i, 0)),
        out_shape=jax.ShapeDtypeStruct((4, P, 64), jnp.float32),
    )(x, W1, degp)

    s1 = _make_prop(64, 4)(t1, src_pr, dst_pr)

    t2 = pl.pallas_call(
        _mm2_body,
        grid=(P // BLK,),
        in_specs=[
            pl.BlockSpec((4, BLK, 64), lambda i: (0, i, 0)),
            pl.BlockSpec((2, BLK, 16), lambda i: (0, i, 0)),
            pl.BlockSpec((4, 1, 64), lambda i: (0, 0, 0)),
            pl.BlockSpec((4, 64, CC), lambda i: (0, 0, 0)),
        ],
        out_specs=pl.BlockSpec((2, BLK, 32), lambda i: (0, i, 0)),
        out_shape=jax.ShapeDtypeStruct((2, P, 32), jnp.float32),
    )(s1, degp, b1r, W2r)

    s2 = _make_prop(32, 2)(t2, src_pr, dst_pr)

    out = pl.pallas_call(
        _lsm_body,
        grid=(P // BLK,),
        in_specs=[
            pl.BlockSpec((2, BLK, 32), lambda i: (0, i, 0)),
            pl.BlockSpec((2, BLK, 16), lambda i: (0, i, 0)),
            pl.BlockSpec((2, 1, 32), lambda i: (0, 0, 0)),
        ],
        out_specs=pl.BlockSpec((BLK, CC), lambda i: (i, 0)),
        out_shape=jax.ShapeDtypeStruct((P, CC), jnp.float32),
    )(s2, degp, b2r)

    return out[:NN]


# speedup vs baseline: 1.1459x; 1.1122x over previous
import functools

import jax
import jax.numpy as jnp
from jax import lax
from jax.experimental import pallas as pl
from jax.experimental.pallas import tpu as pltpu
from jax.experimental.pallas import tpu_sc as plsc

NN = 10000
EE = 320000
FIN = 128
HID = 256
CC = 64

P = 10240
ROWS = P // 16
K = 128
NB = 160
EP = 16 * NB * K
NBD = 80
NBUF = 5
BLK = 1024


@functools.cache
def _mesh():
    return plsc.VectorSubcoreMesh(core_axis_name="c", subcore_axis_name="s",
                                  num_cores=2, num_subcores=16)


@functools.cache
def _make_deg():
    @functools.partial(
        pl.kernel,
        out_type=jax.ShapeDtypeStruct((2, P, 16), jnp.float32),
        mesh=_mesh(),
        compiler_params=pltpu.CompilerParams(use_tc_tiling_on_sc=False),
        scratch_types=[
            pltpu.VMEM((NBD, K), jnp.int32),
            pltpu.VMEM((K, 16), jnp.float32),
            pltpu.VMEM((ROWS, 16), jnp.float32),
            pltpu.VMEM_SHARED((P, 16), jnp.float32),
        ],
    )
    def deg_kernel(dst_hbm, out_hbm, didx, ones_v, zbuf, accum):
        c = lax.axis_index("c")
        s = lax.axis_index("s")
        pltpu.sync_copy(dst_hbm.at[s, pl.ds(c * NBD, NBD)], didx)
        one16 = jnp.ones((16,), jnp.float32)
        zero16 = jnp.zeros((16,), jnp.float32)

        def initbufs(i, carry):
            @pl.when(i < K)
            def _():
                ones_v[i, :] = one16
            zbuf[i, :] = zero16
            return carry

        lax.fori_loop(0, ROWS, initbufs, 0)
        pltpu.sync_copy(zbuf, accum.at[pl.ds(s * ROWS, ROWS)])
        plsc.subcore_barrier()

        def body(j, carry):
            pltpu.sync_copy(ones_v, accum.at[didx.at[j]], add=True)
            return carry

        lax.fori_loop(0, NBD, body, 0)
        plsc.subcore_barrier()
        pltpu.sync_copy(accum.at[pl.ds(s * ROWS, ROWS)],
                        out_hbm.at[c, pl.ds(s * ROWS, ROWS)])

    return deg_kernel


@functools.cache
def _make_prop(F, NCH):
    per_core = NCH // 2

    @functools.partial(
        pl.kernel,
        out_type=jax.ShapeDtypeStruct((NCH, P, F), jnp.float32),
        mesh=_mesh(),
        compiler_params=pltpu.CompilerParams(use_tc_tiling_on_sc=False),
        scratch_types=[
            pltpu.VMEM((NB, K), jnp.int32),
            pltpu.VMEM((NB, K), jnp.int32),
            [pltpu.VMEM((K, F), jnp.float32)] * NBUF,
            [pltpu.SemaphoreType.DMA] * NBUF,
            pltpu.VMEM_SHARED((P, F), jnp.float32),
        ],
    )
    def prop(t_hbm, src_hbm, dst_hbm, out_hbm, sidx, didx, rows, gsems,
             accum):
        c = lax.axis_index("c")
        s = lax.axis_index("s")
        ngrp = NB // NBUF
        pltpu.sync_copy(dst_hbm.at[s], didx)
        pltpu.sync_copy(src_hbm.at[s], sidx)
        for cc in range(per_core):
            ch = c * per_core + cc
            tch = t_hbm.at[ch]
            pltpu.sync_copy(tch.at[pl.ds(s * ROWS, ROWS)],
                            accum.at[pl.ds(s * ROWS, ROWS)])
            plsc.subcore_barrier()

            for b in range(NBUF):
                pltpu.async_copy(tch.at[sidx.at[b]], rows[b], gsems[b])

            def group(g, carry):
                for b in range(NBUF):
                    j = g * NBUF + b
                    pltpu.make_async_copy(tch.at[sidx.at[j]],
                                          rows[b], gsems[b]).wait()
                    pltpu.sync_copy(rows[b], accum.at[didx.at[j]], add=True)

                    @pl.when(g + 1 < ngrp)
                    def _():
                        pltpu.async_copy(tch.at[sidx.at[j + NBUF]],
                                         rows[b], gsems[b])
                return carry

            lax.fori_loop(0, ngrp, group, 0)
            plsc.subcore_barrier()
            pltpu.sync_copy(accum.at[pl.ds(s * ROWS, ROWS)],
                            out_hbm.at[ch, pl.ds(s * ROWS, ROWS)])

    return prop


def _mm1_body(x_ref, w_ref, degp_ref, o_ref):
    deg = degp_ref[0, :, 0:1] + degp_ref[1, :, 0:1] + 1.0
    dinv = lax.rsqrt(deg)
    h = jnp.dot(x_ref[...], w_ref[...], preferred_element_type=jnp.float32)
    hs = h * dinv
    for i in range(4):
        o_ref[i] = hs[:, 64 * i:64 * (i + 1)]


def _mm2_body(s1_ref, degp_ref, b1_ref, w2_ref, o_ref):
    deg = degp_ref[0, :, 0:1] + degp_ref[1, :, 0:1] + 1.0
    dinv = lax.rsqrt(deg)
    acc = None
    for i in range(4):
        z = jnp.maximum(s1_ref[i] * dinv + b1_ref[i], 0.0)
        zi = jnp.dot(z, w2_ref[i], preferred_element_type=jnp.float32)
        acc = zi if acc is None else acc + zi
    hs2 = acc * dinv
    o_ref[0] = hs2[:, :32]
    o_ref[1] = hs2[:, 32:]


def _lsm_body(s2_ref, degp_ref, b2_ref, o_ref):
    deg = degp_ref[0, :, 0:1] + degp_ref[1, :, 0:1] + 1.0
    dinv = lax.rsqrt(deg)
    z0 = s2_ref[0] * dinv + b2_ref[0]
    z1 = s2_ref[1] * dinv + b2_ref[1]
    m = jnp.maximum(jnp.max(z0, axis=1, keepdims=True),
                    jnp.max(z1, axis=1, keepdims=True))
    se = (jnp.sum(jnp.exp(z0 - m), axis=1, keepdims=True)
          + jnp.sum(jnp.exp(z1 - m), axis=1, keepdims=True))
    lse = m + jnp.log(se)
    o_ref[:, :32] = z0 - lse
    o_ref[:, 32:] = z1 - lse


def kernel(x, edge_index, W1, b1, W2, b2):
    src = edge_index[0].astype(jnp.int32)
    dst = edge_index[1].astype(jnp.int32)
    pad = EP - EE
    pad_idx = NN + (jnp.arange(pad, dtype=jnp.int32) % (P - NN))
    src_pr = jnp.concatenate([src, pad_idx]).reshape(16, NB, K)
    dst_pr = jnp.concatenate([dst, pad_idx]).reshape(16, NB, K)
    b1r = b1.reshape(4, 1, 64)
    b2r = b2.reshape(2, 1, 32)
    W2r = W2.reshape(4, 64, CC)

    degp = _make_deg()(dst_pr)

    t1 = pl.pallas_call(
        _mm1_body,
        grid=(NN // 1000,),
        in_specs=[
            pl.BlockSpec((1000, FIN), lambda i: (i, 0)),
            pl.BlockSpec((FIN, HID), lambda i: (0, 0)),
            pl.BlockSpec((2, 1000, 16), lambda i: (0, i, 0)),
        ],
        out_specs=pl.BlockSpec((4, 1000, 64), lambda i: (0, i, 0)),
        out_shape=jax.ShapeDtypeStruct((4, P, 64), jnp.float32),
    )(x, W1, degp)

    s1 = _make_prop(64, 4)(t1, src_pr, dst_pr)

    t2 = pl.pallas_call(
        _mm2_body,
        grid=(P // BLK,),
        in_specs=[
            pl.BlockSpec((4, BLK, 64), lambda i: (0, i, 0)),
            pl.BlockSpec((2, BLK, 16), lambda i: (0, i, 0)),
            pl.BlockSpec((4, 1, 64), lambda i: (0, 0, 0)),
            pl.BlockSpec((4, 64, CC), lambda i: (0, 0, 0)),
        ],
        out_specs=pl.BlockSpec((2, BLK, 32), lambda i: (0, i, 0)),
        out_shape=jax.ShapeDtypeStruct((2, P, 32), jnp.float32),
    )(s1, degp, b1r, W2r)

    s2 = _make_prop(32, 2)(t2, src_pr, dst_pr)

    out = pl.pallas_call(
        _lsm_body,
        grid=(P // BLK,),
        in_specs=[
            pl.BlockSpec((2, BLK, 32), lambda i: (0, i, 0)),
            pl.BlockSpec((2, BLK, 16), lambda i: (0, i, 0)),
            pl.BlockSpec((2, 1, 32), lambda i: (0, 0, 0)),
        ],
        out_specs=pl.BlockSpec((BLK, CC), lambda i: (i, 0)),
        out_shape=jax.ShapeDtypeStruct((P, CC), jnp.float32),
    )(s2, degp, b2r)

    return out[:NN]


# speedup vs baseline: 1.1918x; 1.0400x over previous
import functools

import jax
import jax.numpy as jnp
from jax import lax
from jax.experimental import pallas as pl
from jax.experimental.pallas import tpu as pltpu
from jax.experimental.pallas import tpu_sc as plsc

NN = 10000
EE = 320000
FIN = 128
HID = 256
CC = 64

P = 10240
ROWS = P // 16
K = 128
ET = EE // 16
NBF = ET // K
TK = ET - NBF * K
ETD = ET // 2
NBD = ETD // K
TKD = ETD - NBD * K
NBUF = 6
BLK = 1024


@functools.cache
def _mesh():
    return plsc.VectorSubcoreMesh(core_axis_name="c", subcore_axis_name="s",
                                  num_cores=2, num_subcores=16)


@functools.cache
def _make_deg():
    @functools.partial(
        pl.kernel,
        out_type=jax.ShapeDtypeStruct((2, P, 16), jnp.float32),
        mesh=_mesh(),
        compiler_params=pltpu.CompilerParams(use_tc_tiling_on_sc=False),
        scratch_types=[
            pltpu.VMEM((ETD,), jnp.int32),
            pltpu.VMEM((K, 16), jnp.float32),
            pltpu.VMEM((ROWS, 16), jnp.float32),
            pltpu.VMEM_SHARED((P, 16), jnp.float32),
        ],
    )
    def deg_kernel(ei_hbm, out_hbm, didx, ones_v, zbuf, accum):
        c = lax.axis_index("c")
        s = lax.axis_index("s")
        pltpu.sync_copy(ei_hbm.at[1, pl.ds(s * ET + c * ETD, ETD)], didx)
        one16 = jnp.ones((16,), jnp.float32)
        zero16 = jnp.zeros((16,), jnp.float32)

        def initbufs(i, carry):
            @pl.when(i < K)
            def _():
                ones_v[i, :] = one16
            zbuf[i, :] = zero16
            return carry

        lax.fori_loop(0, ROWS, initbufs, 0)
        pltpu.sync_copy(zbuf, accum.at[pl.ds(s * ROWS, ROWS)])
        plsc.subcore_barrier()

        def body(j, carry):
            pltpu.sync_copy(ones_v, accum.at[didx.at[pl.ds(j * K, K)]],
                            add=True)
            return carry

        lax.fori_loop(0, NBD, body, 0)
        pltpu.sync_copy(ones_v.at[pl.ds(0, TKD)],
                        accum.at[didx.at[pl.ds(NBD * K, TKD)]], add=True)
        plsc.subcore_barrier()
        pltpu.sync_copy(accum.at[pl.ds(s * ROWS, ROWS)],
                        out_hbm.at[c, pl.ds(s * ROWS, ROWS)])

    return deg_kernel


@functools.cache
def _make_prop(F, NCH):
    per_core = NCH // 2

    @functools.partial(
        pl.kernel,
        out_type=jax.ShapeDtypeStruct((NCH, P, F), jnp.float32),
        mesh=_mesh(),
        compiler_params=pltpu.CompilerParams(use_tc_tiling_on_sc=False),
        scratch_types=[
            pltpu.VMEM((ET,), jnp.int32),
            pltpu.VMEM((ET,), jnp.int32),
            [pltpu.VMEM((K, F), jnp.float32)] * NBUF,
            [pltpu.SemaphoreType.DMA] * NBUF,
            pltpu.VMEM_SHARED((P, F), jnp.float32),
        ],
    )
    def prop(t_hbm, ei_hbm, out_hbm, sidx, didx, rows, gsems, accum):
        c = lax.axis_index("c")
        s = lax.axis_index("s")
        ngrp = NBF // NBUF
        pltpu.sync_copy(ei_hbm.at[0, pl.ds(s * ET, ET)], sidx)
        pltpu.sync_copy(ei_hbm.at[1, pl.ds(s * ET, ET)], didx)
        for cc in range(per_core):
            ch = c * per_core + cc
            tch = t_hbm.at[ch]
            pltpu.sync_copy(tch.at[pl.ds(s * ROWS, ROWS)],
                            accum.at[pl.ds(s * ROWS, ROWS)])
            plsc.subcore_barrier()

            for b in range(NBUF):
                pltpu.async_copy(tch.at[sidx.at[pl.ds(b * K, K)]],
                                 rows[b], gsems[b])

            def group(g, carry):
                for b in range(NBUF):
                    j = g * NBUF + b
                    pltpu.make_async_copy(tch.at[sidx.at[pl.ds(j * K, K)]],
                                          rows[b], gsems[b]).wait()
                    pltpu.sync_copy(rows[b],
                                    accum.at[didx.at[pl.ds(j * K, K)]],
                                    add=True)

                    @pl.when(g + 1 < ngrp)
                    def _():
                        pltpu.async_copy(
                            tch.at[sidx.at[pl.ds((j + NBUF) * K, K)]],
                            rows[b], gsems[b])
                return carry

            lax.fori_loop(0, ngrp, group, 0)
            pltpu.async_copy(tch.at[sidx.at[pl.ds(NBF * K, TK)]],
                             rows[0].at[pl.ds(0, TK)], gsems[0]).wait()
            pltpu.sync_copy(rows[0].at[pl.ds(0, TK)],
                            accum.at[didx.at[pl.ds(NBF * K, TK)]], add=True)
            plsc.subcore_barrier()
            pltpu.sync_copy(accum.at[pl.ds(s * ROWS, ROWS)],
                            out_hbm.at[ch, pl.ds(s * ROWS, ROWS)])

    return prop


def _mm1_body(x_ref, w_ref, degp_ref, o_ref):
    deg = degp_ref[0, :, 0:1] + degp_ref[1, :, 0:1] + 1.0
    dinv = lax.rsqrt(deg)
    h = jnp.dot(x_ref[...], w_ref[...], preferred_element_type=jnp.float32)
    hs = h * dinv
    for i in range(4):
        o_ref[i] = hs[:, 64 * i:64 * (i + 1)]


def _mm2_body(s1_ref, degp_ref, b1_ref, w2_ref, o_ref):
    deg = degp_ref[0, :, 0:1] + degp_ref[1, :, 0:1] + 1.0
    dinv = lax.rsqrt(deg)
    acc = None
    for i in range(4):
        z = jnp.maximum(s1_ref[i] * dinv + b1_ref[i], 0.0)
        zi = jnp.dot(z, w2_ref[i], preferred_element_type=jnp.float32)
        acc = zi if acc is None else acc + zi
    hs2 = acc * dinv
    o_ref[0] = hs2[:, :32]
    o_ref[1] = hs2[:, 32:]


def _lsm_body(s2_ref, degp_ref, b2_ref, o_ref):
    deg = degp_ref[0, :, 0:1] + degp_ref[1, :, 0:1] + 1.0
    dinv = lax.rsqrt(deg)
    z0 = s2_ref[0] * dinv + b2_ref[0]
    z1 = s2_ref[1] * dinv + b2_ref[1]
    m = jnp.maximum(jnp.max(z0, axis=1, keepdims=True),
                    jnp.max(z1, axis=1, keepdims=True))
    se = (jnp.sum(jnp.exp(z0 - m), axis=1, keepdims=True)
          + jnp.sum(jnp.exp(z1 - m), axis=1, keepdims=True))
    lse = m + jnp.log(se)
    o_ref[:, :32] = z0 - lse
    o_ref[:, 32:] = z1 - lse


def kernel(x, edge_index, W1, b1, W2, b2):
    ei = edge_index.astype(jnp.int32)
    b1r = b1.reshape(4, 1, 64)
    b2r = b2.reshape(2, 1, 32)
    W2r = W2.reshape(4, 64, CC)

    degp = _make_deg()(ei)

    t1 = pl.pallas_call(
        _mm1_body,
        grid=(NN // 1000,),
        in_specs=[
            pl.BlockSpec((1000, FIN), lambda i: (i, 0)),
            pl.BlockSpec((FIN, HID), lambda i: (0, 0)),
            pl.BlockSpec((2, 1000, 16), lambda i: (0, i, 0)),
        ],
        out_specs=pl.BlockSpec((4, 1000, 64), lambda i: (0, i, 0)),
        out_shape=jax.ShapeDtypeStruct((4, P, 64), jnp.float32),
    )(x, W1, degp)

    s1 = _make_prop(64, 4)(t1, ei)

    t2 = pl.pallas_call(
        _mm2_body,
        grid=(P // BLK,),
        in_specs=[
            pl.BlockSpec((4, BLK, 64), lambda i: (0, i, 0)),
            pl.BlockSpec((2, BLK, 16), lambda i: (0, i, 0)),
            pl.BlockSpec((4, 1, 64), lambda i: (0, 0, 0)),
            pl.BlockSpec((4, 64, CC), lambda i: (0, 0, 0)),
        ],
        out_specs=pl.BlockSpec((2, BLK, 32), lambda i: (0, i, 0)),
        out_shape=jax.ShapeDtypeStruct((2, P, 32), jnp.float32),
    )(s1, degp, b1r, W2r)

    s2 = _make_prop(32, 2)(t2, ei)

    out = pl.pallas_call(
        _lsm_body,
        grid=(NN // 1000,),
        in_specs=[
            pl.BlockSpec((2, 1000, 32), lambda i: (0, i, 0)),
            pl.BlockSpec((2, 1000, 16), lambda i: (0, i, 0)),
            pl.BlockSpec((2, 1, 32), lambda i: (0, 0, 0)),
        ],
        out_specs=pl.BlockSpec((1000, CC), lambda i: (i, 0)),
        out_shape=jax.ShapeDtypeStruct((NN, CC), jnp.float32),
    )(s2, degp, b2r)

    return out
